# baseline (speedup 1.0000x reference)
import functools

import jax
import jax.numpy as jnp
from jax import lax
from jax.experimental import pallas as pl
from jax.experimental.pallas import tpu as pltpu
from jax.experimental.pallas import tpu_sc as plsc

D_MODEL = 64
PAIR_W = 2 * D_MODEL
SEQ = 200
PAIRS = SEQ // 2
IDX_W = 128
NBUF = 2
N_WORKERS = 32
N_CORES = 2
LANES = 16


def _emb_kernel(n_batch, n_vocab):
    n_elems = n_batch // N_WORKERS
    assert n_elems % NBUF == 0
    mesh = plsc.VectorSubcoreMesh(core_axis_name="c", subcore_axis_name="s")

    @functools.partial(
        pl.kernel,
        mesh=mesh,
        out_type=jax.ShapeDtypeStruct((n_batch, SEQ, D_MODEL), jnp.float32),
        scratch_types=[
            pltpu.VMEM((1, n_elems, IDX_W), jnp.int32),
            pltpu.VMEM((NBUF, PAIRS, PAIR_W), jnp.float32),
            pltpu.VMEM((NBUF, 1, SEQ, D_MODEL), jnp.float32),
            pltpu.VMEM_SHARED((n_vocab * n_vocab, PAIR_W), jnp.float32),
            pltpu.SemaphoreType.DMA((NBUF,)),
            pltpu.SemaphoreType.DMA((NBUF,)),
        ],
    )
    def emb(idx_hbm, table_hbm, out_hbm, idx_v, pair_v, rows_v, table_sh,
            gsem, osem):
        wid = lax.axis_index("s") * N_CORES + lax.axis_index("c")
        elem_base = wid * n_elems

        @pl.when(lax.axis_index("s") == 0)
        def _():
            pltpu.sync_copy(table_hbm, table_sh)

        pltpu.sync_copy(idx_hbm.at[pl.ds(wid, 1)], idx_v)
        plsc.subcore_barrier()

        def start_gather(e, b):
            pltpu.async_copy(
                table_sh.at[idx_v.at[0, e, pl.ds(0, PAIRS)]], pair_v.at[b],
                gsem.at[b])

        def wait_gather(e, b):
            pltpu.make_async_copy(
                table_sh.at[idx_v.at[0, e, pl.ds(0, PAIRS)]], pair_v.at[b],
                gsem.at[b]).wait()

        def strip(b):
            for r in range(PAIRS):
                for c in range(0, D_MODEL, LANES):
                    rows_v[b, 0, 2 * r, pl.ds(c, LANES)] = (
                        pair_v[b, r, pl.ds(c, LANES)])
                    rows_v[b, 0, 2 * r + 1, pl.ds(c, LANES)] = (
                        pair_v[b, r, pl.ds(D_MODEL + c, LANES)])

        def start_out(e, b):
            pltpu.async_copy(
                rows_v.at[b], out_hbm.at[pl.ds(elem_base + e, 1)], osem.at[b])

        def wait_out(e, b):
            pltpu.make_async_copy(
                rows_v.at[b], out_hbm.at[pl.ds(elem_base + e, 1)], osem.at[b]
            ).wait()

        start_gather(0, 0)

        def group(g, carry):
            eo = g * NBUF
            for b in range(NBUF):
                e = eo + b
                wait_gather(e, b)

                @pl.when(e + 1 < n_elems)
                def _():
                    start_gather(e + 1, (b + 1) % NBUF)

                @pl.when(e >= NBUF)
                def _():
                    wait_out(e - NBUF, b)

                strip(b)
                start_out(e, b)

            return carry

        lax.fori_loop(0, n_elems // NBUF, group, 0)

        for e in range(n_elems - NBUF, n_elems):
            wait_out(e, e % NBUF)

    return emb


def kernel(edge_type, op_embedding):
    b0, b1 = edge_type.shape
    v = op_embedding.shape[0]
    flat = edge_type.reshape(-1).astype(jnp.int32)
    pair_idx = (flat[0::2] * v + flat[1::2]).reshape(b0, PAIRS)
    pair_idx = jnp.pad(pair_idx, ((0, 0), (0, IDX_W - PAIRS)))
    pair_idx = pair_idx.reshape(N_WORKERS, -1, IDX_W)
    table = op_embedding.astype(jnp.float32)
    ptable = jnp.concatenate(
        [
            jnp.broadcast_to(table[:, None, :], (v, v, D_MODEL)),
            jnp.broadcast_to(table[None, :, :], (v, v, D_MODEL)),
        ],
        axis=-1,
    ).reshape(v * v, PAIR_W)
    return _emb_kernel(b0, v)(pair_idx, ptable)
